# Initial kernel scaffold; baseline (speedup 1.0000x reference)
import jax
import jax.numpy as jnp
from jax.experimental import pallas as pl


def kernel(position_ids, table):
    raise NotImplementedError("write your pallas kernel here")


# speedup vs baseline: 1.5989x; 1.5989x over previous
import functools

import jax
import jax.numpy as jnp
from jax import lax
from jax.experimental import pallas as pl
from jax.experimental.pallas import tpu as pltpu
from jax.experimental.pallas import tpu_sc as plsc

SEQ = 8192
DIM = 2048
TOT = 4 * 8192
NC, NS = 2, 16
NW = NC * NS
PER_W = TOT // NW
CHUNK = 16
NCHUNK = PER_W // CHUNK
NPAIR = NCHUNK // 2

_mesh = plsc.VectorSubcoreMesh(core_axis_name="c", subcore_axis_name="s")


@functools.partial(
    pl.kernel,
    out_type=jax.ShapeDtypeStruct((TOT, DIM), jnp.float32),
    mesh=_mesh,
    scratch_types=[
        pltpu.VMEM((NCHUNK, CHUNK), jnp.int32),
        pltpu.VMEM((CHUNK, DIM), jnp.float32),
        pltpu.VMEM((CHUNK, DIM), jnp.float32),
        pltpu.SemaphoreType.DMA,
        pltpu.SemaphoreType.DMA,
    ],
)
def _gather_sc(ids_hbm, table_hbm, out_hbm, idx_v, buf0, buf1, sem0, sem1):
    wid = lax.axis_index("s") * NC + lax.axis_index("c")
    base = wid * PER_W

    pltpu.sync_copy(ids_hbm.at[wid], idx_v)

    def gather(j, buf, sem):
        return pltpu.async_copy(table_hbm.at[idx_v.at[j]], buf, sem)

    def wait(buf, sem):
        pltpu.make_async_copy(table_hbm.at[idx_v.at[0]], buf, sem).wait()

    def put(j, buf):
        pltpu.sync_copy(buf, out_hbm.at[pl.ds(base + j * CHUNK, CHUNK)])

    gather(0, buf0, sem0)
    gather(1, buf1, sem1)

    def body(i, carry):
        j = i * 2
        wait(buf0, sem0)
        put(j, buf0)
        gather(j + 2, buf0, sem0)
        wait(buf1, sem1)
        put(j + 1, buf1)
        gather(j + 3, buf1, sem1)
        return carry

    lax.fori_loop(0, NPAIR - 1, body, 0)

    j = (NPAIR - 1) * 2
    wait(buf0, sem0)
    put(j, buf0)
    wait(buf1, sem1)
    put(j + 1, buf1)


def kernel(position_ids, table):
    ids = position_ids.reshape(NW, NCHUNK, CHUNK).astype(jnp.int32)
    out = _gather_sc(ids, table)
    return out.reshape(position_ids.shape[0], position_ids.shape[1], DIM)
